# baseline (speedup 1.0000x reference)
import functools

import jax
import jax.numpy as jnp
from jax import lax
from jax.experimental import pallas as pl
from jax.experimental.pallas import tpu as pltpu
from jax.experimental.pallas import tpu_sc as plsc

N_STU = 1000000
OUT_DIM = 32
BATCH = 16384
HIST = 50

NC = 2
NS = 16
NW = NC * NS
L = 16

TOTAL = BATCH * HIST
PER_W = TOTAL // NW
CHUNK = 1024
N_CHUNKS = PER_W // CHUNK


def _body(sid_hbm, emb_hbm, out_hbm, idx_v, rows_v, sem):
    wid = lax.axis_index("s") * NC + lax.axis_index("c")
    base = wid * PER_W
    pltpu.sync_copy(sid_hbm.at[pl.ds(base, PER_W)], idx_v)

    def chunk_step(j, carry):
        off = pl.multiple_of(j * CHUNK, 8)
        pltpu.async_copy(emb_hbm.at[idx_v.at[pl.ds(off, CHUNK)]], rows_v, sem).wait()

        def row_step(r, c):
            for h in (0, 16):
                x = rows_v[r, pl.ds(h, L)]
                rows_v[r, pl.ds(h, L)] = 1.0 / (1.0 + jnp.exp(-x))
            return c

        lax.fori_loop(0, CHUNK, row_step, 0, unroll=2)
        pltpu.sync_copy(rows_v, out_hbm.at[pl.ds(base + off, CHUNK)])
        return carry

    lax.fori_loop(0, N_CHUNKS, chunk_step, 0)


@jax.jit
def _run(sid_flat, emb):
    mesh = plsc.VectorSubcoreMesh(core_axis_name="c", subcore_axis_name="s")
    f = pl.kernel(
        _body,
        out_type=jax.ShapeDtypeStruct((TOTAL, OUT_DIM), jnp.float32),
        mesh=mesh,
        scratch_types=[
            pltpu.VMEM((PER_W,), jnp.int32),
            pltpu.VMEM((CHUNK, OUT_DIM), jnp.float32),
            pltpu.SemaphoreType.DMA,
        ],
        compiler_params=pltpu.CompilerParams(use_tc_tiling_on_sc=False),
    )
    return f(sid_flat, emb)


def kernel(sid, emb):
    sid_flat = sid.reshape(-1).astype(jnp.int32)
    out = _run(sid_flat, emb)
    return out.reshape(BATCH, HIST, OUT_DIM)


# speedup vs baseline: 1.1385x; 1.1385x over previous
import jax
import jax.numpy as jnp
from jax import lax
from jax.experimental import pallas as pl
from jax.experimental.pallas import tpu as pltpu
from jax.experimental.pallas import tpu_sc as plsc

N_STU = 1000000
OUT_DIM = 32
BATCH = 16384
HIST = 50

NC = 2
NS = 16
NW = NC * NS
L = 16

TOTAL = BATCH * HIST
PER_W = TOTAL // NW
CHUNK = 512
N_CHUNKS = PER_W // CHUNK
N_PAIRS = N_CHUNKS // 2


def _body(sid_hbm, emb_hbm, out_hbm, idx_v, rows0, rows1, sem0, sem1):
    wid = lax.axis_index("s") * NC + lax.axis_index("c")
    base = wid * PER_W
    pltpu.sync_copy(sid_hbm.at[pl.ds(base, PER_W)], idx_v)

    bufs = (rows0, rows1)
    sems = (sem0, sem1)

    def gather_src(j):
        off = pl.multiple_of(j * CHUNK, 8)
        return emb_hbm.at[idx_v.at[pl.ds(off, CHUNK)]]

    pltpu.async_copy(gather_src(0), rows0, sem0)
    pltpu.async_copy(gather_src(1), rows1, sem1)

    def pair_step(jj, carry):
        for b in (0, 1):
            j = jj * 2 + b
            buf, sem = bufs[b], sems[b]
            pltpu.make_async_copy(gather_src(j), buf, sem).wait()

            @plsc.parallel_loop(0, CHUNK, unroll=8)
            def _sig(r):
                for h in (0, L):
                    x = buf[r, pl.ds(h, L)]
                    buf[r, pl.ds(h, L)] = 1.0 / (1.0 + jnp.exp(-x))

            off = pl.multiple_of(j * CHUNK, 8)
            pltpu.sync_copy(buf, out_hbm.at[pl.ds(base + off, CHUNK)])

            nj = j + 2

            @pl.when(nj < N_CHUNKS)
            def _():
                pltpu.async_copy(gather_src(nj), buf, sem)

        return carry

    lax.fori_loop(0, N_PAIRS, pair_step, 0)


@jax.jit
def _run(sid_flat, emb):
    mesh = plsc.VectorSubcoreMesh(core_axis_name="c", subcore_axis_name="s")
    f = pl.kernel(
        _body,
        out_type=jax.ShapeDtypeStruct((TOTAL, OUT_DIM), jnp.float32),
        mesh=mesh,
        scratch_types=[
            pltpu.VMEM((PER_W,), jnp.int32),
            pltpu.VMEM((CHUNK, OUT_DIM), jnp.float32),
            pltpu.VMEM((CHUNK, OUT_DIM), jnp.float32),
            pltpu.SemaphoreType.DMA,
            pltpu.SemaphoreType.DMA,
        ],
        compiler_params=pltpu.CompilerParams(use_tc_tiling_on_sc=False),
    )
    return f(sid_flat, emb)


def kernel(sid, emb):
    sid_flat = sid.reshape(-1).astype(jnp.int32)
    out = _run(sid_flat, emb)
    return out.reshape(BATCH, HIST, OUT_DIM)


# speedup vs baseline: 1.5499x; 1.3613x over previous
import jax
import jax.numpy as jnp
from jax import lax
from jax.experimental import pallas as pl
from jax.experimental.pallas import tpu as pltpu
from jax.experimental.pallas import tpu_sc as plsc

N_STU = 1000000
OUT_DIM = 32
BATCH = 16384
HIST = 50

NC = 2
NS = 16
NW = NC * NS
L = 16

W = 256
BLOCKS_PER_H = BATCH // W
N_BLOCKS = HIST * BLOCKS_PER_H
PER_W = N_BLOCKS // NW


def _body(sidF, embP, out5, idxr, pbuf, cbuf, gbuf, obuf, sem):
    wid = lax.axis_index("s") * NC + lax.axis_index("c")

    def step(t, carry):
        u = wid * PER_W + t
        h = lax.shift_right_logical(u, 6)
        jb = lax.bitwise_and(u, BLOCKS_PER_H - 1)
        pltpu.sync_copy(sidF.at[pl.ds(u * W, W)], idxr)
        for g in range(16):
            v = idxr[pl.ds(g * L, L)]
            pbuf[g // 8, pl.ds((g % 8) * L, L)] = lax.shift_right_logical(v, 2)
            cbuf[pl.ds(g * L, L)] = lax.shift_left(lax.bitwise_and(v, 3), 5)
        for k in (0, 1):
            pltpu.async_copy(embP.at[pbuf.at[k]], gbuf.at[pl.ds(k * 128, 128)], sem)
        for k in (0, 1):
            pltpu.make_async_copy(embP.at[pbuf.at[k]],
                                  gbuf.at[pl.ds(k * 128, 128)], sem).wait()
        for g in range(16):
            cvec = cbuf[pl.ds(g * L, L)]
            rvec = lax.iota(jnp.int32, L) + g * L
            k = g // 8
            sl = (g % 8) * L

            @plsc.parallel_loop(0, OUT_DIM, unroll=4)
            def _sig(d):
                x = plsc.load_gather(gbuf, [rvec, cvec + d])
                d_hi = lax.shift_right_logical(d, 3)
                d_lo = lax.bitwise_and(d, 7)
                obuf[k, d_hi, d_lo, pl.ds(sl, L)] = 1.0 / (1.0 + jnp.exp(-x))
        for k in (0, 1):
            pltpu.async_copy(obuf.at[k], out5.at[h, :, jb * 2 + k, :, :], sem)
            pltpu.make_async_copy(obuf.at[k],
                                  out5.at[h, :, jb * 2 + k, :, :], sem).wait()
        return carry

    lax.fori_loop(0, PER_W, step, 0)


@jax.jit
def _run(sidF, embP):
    mesh = plsc.VectorSubcoreMesh(core_axis_name="c", subcore_axis_name="s")
    f = pl.kernel(
        _body,
        out_type=jax.ShapeDtypeStruct((HIST, 4, BATCH // 128, 8, 128), jnp.float32),
        mesh=mesh,
        scratch_types=[
            pltpu.VMEM((W,), jnp.int32),
            pltpu.VMEM((2, 128), jnp.int32),
            pltpu.VMEM((W,), jnp.int32),
            pltpu.VMEM((W, 128), jnp.float32),
            pltpu.VMEM((2, 4, 8, 128), jnp.float32),
            pltpu.SemaphoreType.DMA,
        ],
        compiler_params=pltpu.CompilerParams(needs_layout_passes=False),
    )
    return f(sidF, embP)


def kernel(sid, emb):
    sidF = sid.astype(jnp.int32).T.reshape(-1)
    embP = emb.reshape(N_STU // 4, 128)
    out5 = _run(sidF, embP)
    return out5.transpose(2, 4, 0, 1, 3).reshape(BATCH, HIST, OUT_DIM)


# speedup vs baseline: 2.2282x; 1.4376x over previous
import jax
import jax.numpy as jnp
from jax import lax
from jax.experimental import pallas as pl
from jax.experimental.pallas import tpu as pltpu
from jax.experimental.pallas import tpu_sc as plsc

N_STU = 1000000
OUT_DIM = 32
BATCH = 16384
HIST = 50

NC = 2
NS = 16
NW = NC * NS
L = 16

W = 256
BLOCKS_PER_H = BATCH // W
N_BLOCKS = HIST * BLOCKS_PER_H
PER_W = N_BLOCKS // NW


def _body(sidF, embP, out5,
          idxr0, idxr1, pbuf0, pbuf1, cbuf0, cbuf1,
          gbuf0, gbuf1, obuf0, obuf1,
          si0, si1, sg0, sg1, so0, so1):
    wid = lax.axis_index("s") * NC + lax.axis_index("c")
    base = wid * PER_W

    idxr = (idxr0, idxr1)
    pbuf = (pbuf0, pbuf1)
    cbuf = (cbuf0, cbuf1)
    gbuf = (gbuf0, gbuf1)
    obuf = (obuf0, obuf1)
    si = (si0, si1)
    sg = (sg0, sg1)
    so = (so0, so1)

    def idx_start(t, b):
        pltpu.async_copy(sidF.at[pl.ds((base + t) * W, W)], idxr[b], si[b])

    def head(t, b):
        pltpu.make_async_copy(sidF.at[pl.ds((base + t) * W, W)],
                              idxr[b], si[b]).wait()
        for g in range(16):
            v = idxr[b][pl.ds(g * L, L)]
            pbuf[b][g // 8, pl.ds((g % 8) * L, L)] = lax.shift_right_logical(v, 2)
            cbuf[b][pl.ds(g * L, L)] = lax.shift_left(lax.bitwise_and(v, 3), 5)
        for k in (0, 1):
            pltpu.async_copy(embP.at[pbuf[b].at[k]],
                             gbuf[b].at[pl.ds(k * 128, 128)], sg[b])

    def o_wait(b):
        pltpu.make_async_copy(obuf[b].at[0], out5.at[0, :, 0, :, :],
                              so[b]).wait()

    def tail(t, b, drain_out):
        u = base + t
        h = lax.shift_right_logical(u, 6)
        jb = lax.bitwise_and(u, BLOCKS_PER_H - 1)
        for k in (0, 1):
            pltpu.make_async_copy(embP.at[pbuf[b].at[k]],
                                  gbuf[b].at[pl.ds(k * 128, 128)], sg[b]).wait()
        if drain_out:
            o_wait(b)
            o_wait(b)

        @plsc.parallel_loop(0, 16 * OUT_DIM, unroll=4)
        def _sig(i):
            g = lax.shift_right_logical(i, 5)
            d = lax.bitwise_and(i, 31)
            gl = pl.multiple_of(g * L, L)
            cvec = cbuf[b][pl.ds(gl, L)]
            rvec = lax.iota(jnp.int32, L) + gl
            x = plsc.load_gather(gbuf[b], [rvec, cvec + d])
            k = lax.shift_right_logical(g, 3)
            d_hi = lax.shift_right_logical(d, 3)
            d_lo = lax.bitwise_and(d, 7)
            sl = pl.multiple_of(lax.bitwise_and(g, 7) * L, L)
            obuf[b][k, d_hi, d_lo, pl.ds(sl, L)] = 1.0 / (1.0 + jnp.exp(-x))

        for k in (0, 1):
            pltpu.async_copy(obuf[b].at[k], out5.at[h, :, jb * 2 + k, :, :],
                             so[b])

    idx_start(0, 0)
    idx_start(1, 1)
    head(0, 0); idx_start(2, 0)
    head(1, 1); idx_start(3, 1); tail(0, 0, False)
    head(2, 0); idx_start(4, 0); tail(1, 1, False)
    head(3, 1); idx_start(5, 1); tail(2, 0, True)

    def steady(tt, carry):
        t0 = tt * 2
        head(t0, 0); idx_start(t0 + 2, 0); tail(t0 - 1, 1, True)
        head(t0 + 1, 1); idx_start(t0 + 3, 1); tail(t0, 0, True)
        return carry

    lax.fori_loop(2, PER_W // 2 - 1, steady, 0)

    head(PER_W - 2, 0); tail(PER_W - 3, 1, True)
    head(PER_W - 1, 1); tail(PER_W - 2, 0, True)
    tail(PER_W - 1, 1, True)
    o_wait(0); o_wait(0)
    o_wait(1); o_wait(1)


@jax.jit
def _run(sidF, embP):
    mesh = plsc.VectorSubcoreMesh(core_axis_name="c", subcore_axis_name="s")
    f = pl.kernel(
        _body,
        out_type=jax.ShapeDtypeStruct((HIST, 4, BATCH // 128, 8, 128), jnp.float32),
        mesh=mesh,
        scratch_types=[
            pltpu.VMEM((W,), jnp.int32),
            pltpu.VMEM((W,), jnp.int32),
            pltpu.VMEM((2, 128), jnp.int32),
            pltpu.VMEM((2, 128), jnp.int32),
            pltpu.VMEM((W,), jnp.int32),
            pltpu.VMEM((W,), jnp.int32),
            pltpu.VMEM((W, 128), jnp.float32),
            pltpu.VMEM((W, 128), jnp.float32),
            pltpu.VMEM((2, 4, 8, 128), jnp.float32),
            pltpu.VMEM((2, 4, 8, 128), jnp.float32),
            pltpu.SemaphoreType.DMA,
            pltpu.SemaphoreType.DMA,
            pltpu.SemaphoreType.DMA,
            pltpu.SemaphoreType.DMA,
            pltpu.SemaphoreType.DMA,
            pltpu.SemaphoreType.DMA,
        ],
        compiler_params=pltpu.CompilerParams(needs_layout_passes=False),
    )
    return f(sidF, embP)


def kernel(sid, emb):
    sidF = sid.astype(jnp.int32).T.reshape(-1)
    embP = emb.reshape(N_STU // 4, 128)
    out5 = _run(sidF, embP)
    return out5.transpose(2, 4, 0, 1, 3).reshape(BATCH, HIST, OUT_DIM)
